# Initial kernel scaffold; baseline (speedup 1.0000x reference)
import jax
import jax.numpy as jnp
from jax.experimental import pallas as pl


def kernel(x, edge_index, W, b):
    raise NotImplementedError("write your pallas kernel here")


# speedup vs baseline: 2.4723x; 2.4723x over previous
import functools

import jax
import jax.numpy as jnp
from jax import lax
from jax.experimental import pallas as pl
from jax.experimental.pallas import tpu as pltpu
from jax.experimental.pallas import tpu_sc as plsc

N = 10000
D = 128
E = 320000

NC = 2
NS = 16
NW = NC * NS
CH = 128
EPW = 10240
NCHUNK = EPW // CH
E_PAD = EPW * NW
ROWS_PW = 632
N_ACC = ROWS_PW * NS

_sc_mesh = plsc.VectorSubcoreMesh(core_axis_name="c", subcore_axis_name="s")


@functools.partial(
    pl.kernel,
    out_type=jax.ShapeDtypeStruct((NC, N_ACC, D), jnp.float32),
    mesh=_sc_mesh,
    scratch_types=[
        pltpu.VMEM((CH,), jnp.int32),
        pltpu.VMEM((CH,), jnp.int32),
        pltpu.VMEM((CH, D), jnp.float32),
        pltpu.VMEM_SHARED((N_ACC, D), jnp.float32),
        pltpu.SemaphoreType.DMA,
    ],
)
def _gather_scatter_sum(x_hbm, src_hbm, dst_hbm, zeros_hbm, out_hbm,
                        src_v, dst_v, rows_v, acc_sh, sem):
    cid = lax.axis_index("c")
    sid = lax.axis_index("s")
    wid = sid * NC + cid

    r0 = sid * ROWS_PW
    pltpu.sync_copy(zeros_hbm, acc_sh.at[pl.ds(r0, ROWS_PW)])
    plsc.subcore_barrier()

    def body(k, carry):
        base = wid * EPW + k * CH
        pltpu.sync_copy(src_hbm.at[pl.ds(base, CH)], src_v)
        pltpu.sync_copy(dst_hbm.at[pl.ds(base, CH)], dst_v)
        pltpu.async_copy(x_hbm.at[src_v], rows_v, sem).wait()
        pltpu.sync_copy(rows_v, acc_sh.at[dst_v], add=True)
        return carry

    lax.fori_loop(0, NCHUNK, body, 0)
    plsc.subcore_barrier()

    pltpu.sync_copy(acc_sh.at[pl.ds(r0, ROWS_PW)],
                    out_hbm.at[cid, pl.ds(r0, ROWS_PW)])


BM = 400


def _linear_body(parts_ref, w_ref, b_ref, o_ref):
    s = parts_ref[0] + parts_ref[1]
    o_ref[...] = lax.dot_general(
        s, w_ref[...], (((1,), (1,)), ((), ())),
        preferred_element_type=jnp.float32) + b_ref[...]


def _linear(parts, W, b2):
    return pl.pallas_call(
        _linear_body,
        grid=(N // BM,),
        in_specs=[
            pl.BlockSpec((NC, BM, D), lambda i: (0, i, 0)),
            pl.BlockSpec((D, D), lambda i: (0, 0)),
            pl.BlockSpec((1, D), lambda i: (0, 0)),
        ],
        out_specs=pl.BlockSpec((BM, D), lambda i: (i, 0)),
        out_shape=jax.ShapeDtypeStruct((N, D), jnp.float32),
    )(parts, W, b2)


def kernel(x, edge_index, W, b):
    src = edge_index[0]
    dst = edge_index[1]
    pad = E_PAD - E
    src_p = jnp.concatenate([src, jnp.zeros((pad,), jnp.int32)])
    dst_p = jnp.concatenate([dst, jnp.full((pad,), N, jnp.int32)])
    zeros = jnp.zeros((ROWS_PW, D), jnp.float32)
    parts = _gather_scatter_sum(x, src_p, dst_p, zeros)
    h = _linear(parts, W, b.reshape(1, D))
    return h.reshape(1, N, D)


# speedup vs baseline: 3.5979x; 1.4553x over previous
import functools

import jax
import jax.numpy as jnp
from jax import lax
from jax.experimental import pallas as pl
from jax.experimental.pallas import tpu as pltpu
from jax.experimental.pallas import tpu_sc as plsc

N = 10000
D = 128
E = 320000

NC = 2
NS = 16
NW = NC * NS
CH = 96
EPW = 10176
NCHUNK = EPW // CH
E_PAD = EPW * NW
NBUF = 2
NGRP = NCHUNK // NBUF
ROWS_PW = 632
N_ACC = ROWS_PW * NS

_sc_mesh = plsc.VectorSubcoreMesh(core_axis_name="c", subcore_axis_name="s")


@functools.partial(
    pl.kernel,
    out_type=jax.ShapeDtypeStruct((NC, N_ACC, D), jnp.float32),
    mesh=_sc_mesh,
    scratch_types=[
        pltpu.VMEM((EPW,), jnp.int32),
        pltpu.VMEM((NCHUNK, CH), jnp.int32),
        pltpu.VMEM((CH, D), jnp.float32),
        pltpu.VMEM((CH, D), jnp.float32),
        pltpu.VMEM_SHARED((N_ACC, D), jnp.float32),
        pltpu.SemaphoreType.DMA((NBUF,)),
        pltpu.SemaphoreType.DMA((NBUF,)),
    ],
)
def _gather_scatter_sum(x_hbm, src_hbm, dst_hbm, zeros_hbm, out_hbm,
                        src_all, dst_all, buf0, buf1, acc_sh, gsem, ssem):
    bufs = (buf0, buf1)
    cid = lax.axis_index("c")
    sid = lax.axis_index("s")
    wid = sid * NC + cid

    pltpu.sync_copy(src_hbm.at[pl.ds(wid * EPW, EPW)], src_all)
    pltpu.sync_copy(dst_hbm.at[wid], dst_all)
    r0 = sid * ROWS_PW
    pltpu.sync_copy(zeros_hbm, acc_sh.at[pl.ds(r0, ROWS_PW)])
    plsc.subcore_barrier()

    def gather_start(k, b):
        pltpu.async_copy(x_hbm.at[src_all.at[pl.ds(k * CH, CH)]], bufs[b],
                         gsem.at[b])

    def gather_wait(k, b):
        pltpu.make_async_copy(x_hbm.at[src_all.at[pl.ds(k * CH, CH)]],
                              bufs[b], gsem.at[b]).wait()

    def scatter_start(k, b):
        pltpu.async_copy(bufs[b], acc_sh.at[dst_all.at[k]], ssem.at[b],
                         add=True)

    def scatter_wait(k, b):
        pltpu.make_async_copy(bufs[b], acc_sh.at[dst_all.at[k]],
                              ssem.at[b]).wait()

    for b in range(NBUF):
        gather_start(b, b)

    def body(g, carry):
        k0 = g * NBUF
        for b in range(NBUF):
            gather_wait(k0 + b, b)
            scatter_start(k0 + b, b)
        @pl.when(g < NGRP - 1)
        def _():
            for b in range(NBUF):
                scatter_wait(k0 + b, b)
                gather_start(k0 + NBUF + b, b)
        return carry

    lax.fori_loop(0, NGRP, body, 0)
    for b in range(NBUF):
        scatter_wait((NGRP - 1) * NBUF + b, b)
    plsc.subcore_barrier()

    pltpu.sync_copy(acc_sh.at[pl.ds(r0, ROWS_PW)],
                    out_hbm.at[cid, pl.ds(r0, ROWS_PW)])


BM = 400


def _linear_body(parts_ref, w_ref, b_ref, o_ref):
    s = parts_ref[0] + parts_ref[1]
    o_ref[...] = lax.dot_general(
        s, w_ref[...], (((1,), (1,)), ((), ())),
        preferred_element_type=jnp.float32) + b_ref[...]


def _linear(parts, W, b2):
    return pl.pallas_call(
        _linear_body,
        grid=(N // BM,),
        in_specs=[
            pl.BlockSpec((NC, BM, D), lambda i: (0, i, 0)),
            pl.BlockSpec((D, D), lambda i: (0, 0)),
            pl.BlockSpec((1, D), lambda i: (0, 0)),
        ],
        out_specs=pl.BlockSpec((BM, D), lambda i: (i, 0)),
        out_shape=jax.ShapeDtypeStruct((N, D), jnp.float32),
    )(parts, W, b2)


def kernel(x, edge_index, W, b):
    src = edge_index[0]
    dst = edge_index[1]
    pad = E_PAD - E
    src_p = jnp.concatenate([src, jnp.zeros((pad,), jnp.int32)])
    dst_p = jnp.concatenate([dst, jnp.full((pad,), N, jnp.int32)]).reshape(
        NW, NCHUNK, CH)
    zeros = jnp.zeros((ROWS_PW, D), jnp.float32)
    parts = _gather_scatter_sum(x, src_p, dst_p, zeros)
    h = _linear(parts, W, b.reshape(1, D))
    return h.reshape(1, N, D)


# speedup vs baseline: 3.7388x; 1.0392x over previous
import functools

import jax
import jax.numpy as jnp
from jax import lax
from jax.experimental import pallas as pl
from jax.experimental.pallas import tpu as pltpu
from jax.experimental.pallas import tpu_sc as plsc

N = 10000
D = 128
E = 320000

NC = 2
NS = 16
NW = NC * NS
CH = 96
EPW = 10176
NCHUNK = EPW // CH
E_PAD = EPW * NW
NBUF = 2
NGRP = NCHUNK // NBUF
ROWS_PW = 632
N_ACC = ROWS_PW * NS

_sc_mesh = plsc.VectorSubcoreMesh(core_axis_name="c", subcore_axis_name="s")


@functools.partial(
    pl.kernel,
    out_type=jax.ShapeDtypeStruct((NC, N_ACC, D), jnp.float32),
    mesh=_sc_mesh,
    scratch_types=[
        pltpu.VMEM((EPW,), jnp.int32),
        pltpu.VMEM((NCHUNK, CH), jnp.int32),
        pltpu.VMEM((CH, D), jnp.float32),
        pltpu.VMEM((CH, D), jnp.float32),
        pltpu.VMEM_SHARED((N_ACC, D), jnp.float32),
        pltpu.SemaphoreType.DMA((NBUF,)),
        pltpu.SemaphoreType.DMA((NBUF,)),
    ],
)
def _gather_scatter_sum(x_hbm, src_hbm, dst_hbm, zeros_hbm, out_hbm,
                        src_all, dst_all, buf0, buf1, acc_sh, gsem, ssem):
    bufs = (buf0, buf1)
    cid = lax.axis_index("c")
    sid = lax.axis_index("s")
    wid = sid * NC + cid

    pltpu.sync_copy(src_hbm.at[pl.ds(wid * EPW, EPW)], src_all)
    pltpu.sync_copy(dst_hbm.at[wid], dst_all)
    r0 = sid * ROWS_PW
    pltpu.sync_copy(zeros_hbm, acc_sh.at[pl.ds(r0, ROWS_PW)])
    plsc.subcore_barrier()

    def gather_start(k, b):
        pltpu.async_copy(x_hbm.at[src_all.at[pl.ds(k * CH, CH)]], bufs[b],
                         gsem.at[b])

    def gather_wait(k, b):
        pltpu.make_async_copy(x_hbm.at[src_all.at[pl.ds(k * CH, CH)]],
                              bufs[b], gsem.at[b]).wait()

    def scatter_start(k, b):
        pltpu.async_copy(bufs[b], acc_sh.at[dst_all.at[k]], ssem.at[b],
                         add=True)

    def scatter_wait(k, b):
        pltpu.make_async_copy(bufs[b], acc_sh.at[dst_all.at[k]],
                              ssem.at[b]).wait()

    for b in range(NBUF):
        gather_start(b, b)

    def body(g, carry):
        k0 = g * NBUF
        for b in range(NBUF):
            gather_wait(k0 + b, b)
            scatter_start(k0 + b, b)
        @pl.when(g < NGRP - 1)
        def _():
            for b in range(NBUF):
                scatter_wait(k0 + b, b)
                gather_start(k0 + NBUF + b, b)
        return carry

    lax.fori_loop(0, NGRP, body, 0)
    for b in range(NBUF):
        scatter_wait((NGRP - 1) * NBUF + b, b)
    plsc.subcore_barrier()

    pltpu.sync_copy(acc_sh.at[pl.ds(r0, ROWS_PW)],
                    out_hbm.at[cid, pl.ds(r0, ROWS_PW)])


BM = 400


def _linear_body(parts_ref, w_ref, b_ref, o_ref):
    s = parts_ref[0] + parts_ref[1]
    o_ref[...] = lax.dot_general(
        s, w_ref[...], (((1,), (1,)), ((), ())),
        preferred_element_type=jnp.float32) + b_ref[...]


def _linear(parts, W, b2):
    return pl.pallas_call(
        _linear_body,
        grid=(N // BM,),
        in_specs=[
            pl.BlockSpec((NC, BM, D), lambda i: (0, i, 0)),
            pl.BlockSpec((D, D), lambda i: (0, 0)),
            pl.BlockSpec((1, D), lambda i: (0, 0)),
        ],
        out_specs=pl.BlockSpec((BM, D), lambda i: (i, 0)),
        out_shape=jax.ShapeDtypeStruct((N, D), jnp.float32),
    )(parts, W, b2)


def kernel(x, edge_index, W, b):
    src = edge_index[0]
    dst = edge_index[1]
    pad = E_PAD - E
    src_p = jnp.concatenate([src, jnp.zeros((pad,), jnp.int32)])
    ecore = (jnp.arange(E_PAD, dtype=jnp.int32) // EPW) % NC
    src_p = src_p + ecore * N
    x2 = jnp.concatenate([x, x], axis=0)
    dst_p = jnp.concatenate([dst, jnp.full((pad,), N, jnp.int32)]).reshape(
        NW, NCHUNK, CH)
    zeros = jnp.zeros((ROWS_PW, D), jnp.float32)
    parts = _gather_scatter_sum(x2, src_p, dst_p, zeros)
    h = _linear(parts, W, b.reshape(1, D))
    return h.reshape(1, N, D)


# speedup vs baseline: 7.2830x; 1.9479x over previous
import functools

import jax
import jax.numpy as jnp
from jax import lax
from jax.experimental import pallas as pl
from jax.experimental.pallas import tpu as pltpu
from jax.experimental.pallas import tpu_sc as plsc

N = 10000
D = 128
E = 320000

NC = 2
NS = 16
CH = 128
F0 = 111
F1 = 46
E_PAD = NS * (F0 + F1) * CH
R0E = NS * F0 * CH
NIDX = 4
NROW = 2
GMAX = (max(F0, F1) + 5) // 4
ROWS_PW = 632
N_ACC = ROWS_PW * NS

_sc_mesh = plsc.VectorSubcoreMesh(core_axis_name="c", subcore_axis_name="s")


@functools.partial(
    pl.kernel,
    out_type=jax.ShapeDtypeStruct((NC, N_ACC, D), jnp.float32),
    mesh=_sc_mesh,
    scratch_types=[
        pltpu.VMEM((NIDX, CH), jnp.int32),
        pltpu.VMEM((NIDX, CH), jnp.int32),
        pltpu.VMEM((CH, D), jnp.float32),
        pltpu.VMEM((CH, D), jnp.float32),
        pltpu.VMEM_SHARED((N_ACC, D), jnp.float32),
        pltpu.SemaphoreType.DMA((NIDX,)),
        pltpu.SemaphoreType.DMA((NIDX,)),
        pltpu.SemaphoreType.DMA((NROW,)),
        pltpu.SemaphoreType.DMA((NROW,)),
    ],
)
def _gather_scatter_sum(x_hbm, src_hbm, dst_hbm, zeros_hbm, out_hbm,
                        sring, dring, buf0, buf1, acc_sh,
                        issem, idsem, gsem, ssem):
    bufs = (buf0, buf1)
    cid = lax.axis_index("c")
    sid = lax.axis_index("s")
    nch = lax.select(cid == 0, jnp.int32(F0), jnp.int32(F1))
    ebase = lax.select(cid == 0, sid * (F0 * CH), R0E + sid * (F1 * CH))

    r0 = sid * ROWS_PW
    pltpu.sync_copy(zeros_hbm, acc_sh.at[pl.ds(r0, ROWS_PW)])
    plsc.subcore_barrier()

    def sidx_start(k, j):
        pltpu.async_copy(src_hbm.at[pl.ds(ebase + k * CH, CH)], sring.at[j],
                         issem.at[j])

    def sidx_wait(k, j):
        pltpu.make_async_copy(src_hbm.at[pl.ds(ebase + k * CH, CH)],
                              sring.at[j], issem.at[j]).wait()

    def didx_start(k, j):
        pltpu.async_copy(dst_hbm.at[pl.ds(ebase + k * CH, CH)], dring.at[j],
                         idsem.at[j])

    def didx_wait(k, j):
        pltpu.make_async_copy(dst_hbm.at[pl.ds(ebase + k * CH, CH)],
                              dring.at[j], idsem.at[j]).wait()

    def gather_start(j, r):
        pltpu.async_copy(x_hbm.at[sring.at[j]], bufs[r], gsem.at[r])

    def gather_wait(j, r):
        pltpu.make_async_copy(x_hbm.at[sring.at[j]], bufs[r],
                              gsem.at[r]).wait()

    def scatter_start(j, r):
        pltpu.async_copy(bufs[r], acc_sh.at[dring.at[j]], ssem.at[r],
                         add=True)

    def scatter_wait(j, r):
        pltpu.make_async_copy(bufs[r], acc_sh.at[dring.at[j]],
                              ssem.at[r]).wait()

    for j in range(NIDX):
        sidx_start(j, j)
        didx_start(j, j)

    def body(g, carry):
        for u in range(4):
            k = g * 4 + u
            @pl.when(jnp.logical_and(k >= 1, k <= nch))
            def _():
                gather_wait((u - 1) % 4, (u - 1) % 2)
                scatter_start((u - 1) % 4, (u - 1) % 2)
            @pl.when(jnp.logical_and(k >= 1, k + 3 <= nch - 1))
            def _():
                sidx_start(k + 3, (u + 3) % 4)
            @pl.when(jnp.logical_and(k >= 2, k <= nch + 1))
            def _():
                scatter_wait((u - 2) % 4, u % 2)
            @pl.when(jnp.logical_and(k >= 2, k + 2 <= nch - 1))
            def _():
                didx_start(k + 2, (u + 2) % 4)
            @pl.when(k <= nch - 1)
            def _():
                sidx_wait(k, u % 4)
                didx_wait(k, u % 4)
                gather_start(u % 4, u % 2)
        return carry

    lax.fori_loop(0, GMAX, body, 0)
    plsc.subcore_barrier()

    pltpu.sync_copy(acc_sh.at[pl.ds(r0, ROWS_PW)],
                    out_hbm.at[cid, pl.ds(r0, ROWS_PW)])


BM = 400


def _linear_body(parts_ref, w_ref, b_ref, o_ref):
    s = parts_ref[0] + parts_ref[1]
    o_ref[...] = lax.dot_general(
        s, w_ref[...], (((1,), (1,)), ((), ())),
        preferred_element_type=jnp.float32) + b_ref[...]


def _linear(parts, W, b2):
    return pl.pallas_call(
        _linear_body,
        grid=(N // BM,),
        in_specs=[
            pl.BlockSpec((NC, BM, D), lambda i: (0, i, 0)),
            pl.BlockSpec((D, D), lambda i: (0, 0)),
            pl.BlockSpec((1, D), lambda i: (0, 0)),
        ],
        out_specs=pl.BlockSpec((BM, D), lambda i: (i, 0)),
        out_shape=jax.ShapeDtypeStruct((N, D), jnp.float32),
    )(parts, W, b2)


def kernel(x, edge_index, W, b):
    src = edge_index[0]
    dst = edge_index[1]
    pad = E_PAD - E
    src_p = jnp.concatenate([src, jnp.zeros((pad,), jnp.int32)])
    dst_p = jnp.concatenate([dst, jnp.full((pad,), N, jnp.int32)])
    ecore = (jnp.arange(E_PAD, dtype=jnp.int32) >= R0E).astype(jnp.int32)
    src_p = src_p + ecore * N
    x2 = jnp.concatenate([x, x], axis=0)
    zeros = jnp.zeros((ROWS_PW, D), jnp.float32)
    parts = _gather_scatter_sum(x2, src_p, dst_p, zeros)
    h = _linear(parts, W, b.reshape(1, D))
    return h.reshape(1, N, D)


# speedup vs baseline: 8.7015x; 1.1948x over previous
import functools

import jax
import jax.numpy as jnp
from jax import lax
from jax.experimental import pallas as pl
from jax.experimental.pallas import tpu as pltpu
from jax.experimental.pallas import tpu_sc as plsc

N = 10000
D = 128
E = 320000

NC = 2
NS = 16
CH = 96
F0 = 139
F1 = 70
E_PAD = NS * (F0 + F1) * CH
R0E = NS * F0 * CH
GMAX = (max(F0, F1) + 3 + 7) // 8
N_ACC = 10008
ROWS_PW = 632
ROWS_LAST = N_ACC - 15 * ROWS_PW

_sc_mesh = plsc.VectorSubcoreMesh(core_axis_name="c", subcore_axis_name="s")


@functools.partial(
    pl.kernel,
    out_type=jax.ShapeDtypeStruct((NC, N_ACC, D), jnp.float32),
    mesh=_sc_mesh,
    scratch_types=[
        pltpu.VMEM((4, CH), jnp.int32),
        pltpu.VMEM((8, CH), jnp.int32),
        pltpu.VMEM((CH, D), jnp.float32),
        pltpu.VMEM((CH, D), jnp.float32),
        pltpu.VMEM((CH, D), jnp.float32),
        pltpu.VMEM((CH, D), jnp.float32),
        pltpu.VMEM_SHARED((N_ACC, D), jnp.float32),
        pltpu.SemaphoreType.DMA((4,)),
        pltpu.SemaphoreType.DMA((8,)),
        pltpu.SemaphoreType.DMA((4,)),
        pltpu.SemaphoreType.DMA((4,)),
    ],
)
def _gather_scatter_sum(x_hbm, src_hbm, dst_hbm, zeros_hbm, out_hbm,
                        sring, dring, buf0, buf1, buf2, buf3, acc_sh,
                        issem, idsem, gsem, ssem):
    bufs = (buf0, buf1, buf2, buf3)
    cid = lax.axis_index("c")
    sid = lax.axis_index("s")
    nch = lax.select(cid == 0, jnp.int32(F0), jnp.int32(F1))
    ebase = lax.select(cid == 0, sid * (F0 * CH), R0E + sid * (F1 * CH))

    r0 = sid * ROWS_PW

    @pl.when(sid < NS - 1)
    def _():
        pltpu.sync_copy(zeros_hbm, acc_sh.at[pl.ds(r0, ROWS_PW)])

    @pl.when(sid == NS - 1)
    def _():
        pltpu.sync_copy(zeros_hbm.at[pl.ds(0, ROWS_LAST)],
                        acc_sh.at[pl.ds((NS - 1) * ROWS_PW, ROWS_LAST)])

    plsc.subcore_barrier()

    def sidx_start(k, j):
        pltpu.async_copy(src_hbm.at[pl.ds(ebase + k * CH, CH)], sring.at[j],
                         issem.at[j])

    def sidx_wait(k, j):
        pltpu.make_async_copy(src_hbm.at[pl.ds(ebase + k * CH, CH)],
                              sring.at[j], issem.at[j]).wait()

    def didx_start(k, j):
        pltpu.async_copy(dst_hbm.at[pl.ds(ebase + k * CH, CH)], dring.at[j],
                         idsem.at[j])

    def didx_wait(k, j):
        pltpu.make_async_copy(dst_hbm.at[pl.ds(ebase + k * CH, CH)],
                              dring.at[j], idsem.at[j]).wait()

    def gather_start(j, r):
        pltpu.async_copy(x_hbm.at[sring.at[j]], bufs[r], gsem.at[r])

    def gather_wait(j, r):
        pltpu.make_async_copy(x_hbm.at[sring.at[j]], bufs[r],
                              gsem.at[r]).wait()

    def scatter_start(q, r):
        pltpu.async_copy(bufs[r], acc_sh.at[dring.at[q]], ssem.at[r],
                         add=True)

    def scatter_wait(q, r):
        pltpu.make_async_copy(bufs[r], acc_sh.at[dring.at[q]],
                              ssem.at[r]).wait()

    for j in range(4):
        sidx_start(j, j)
        didx_start(j, j)
    sidx_wait(0, 0)
    gather_start(0, 0)

    def body(g, carry):
        for u in range(8):
            k = g * 8 + u
            @pl.when(jnp.logical_and(k >= 3, k <= nch + 2))
            def _():
                scatter_wait((u - 3) % 8, (u - 3) % 4)
            @pl.when(k + 1 <= nch - 1)
            def _():
                sidx_wait(k + 1, (u + 1) % 4)
                gather_start((u + 1) % 4, (u + 1) % 4)
            @pl.when(k <= nch - 1)
            def _():
                gather_wait(u % 4, u % 4)
                didx_wait(k, u % 8)
                scatter_start(u % 8, u % 4)
            @pl.when(k + 4 <= nch - 1)
            def _():
                sidx_start(k + 4, u % 4)
                didx_start(k + 4, (u + 4) % 8)
        return carry

    lax.fori_loop(0, GMAX, body, 0)
    plsc.subcore_barrier()

    @pl.when(sid < NS - 1)
    def _():
        pltpu.sync_copy(acc_sh.at[pl.ds(r0, ROWS_PW)],
                        out_hbm.at[cid, pl.ds(r0, ROWS_PW)])

    @pl.when(sid == NS - 1)
    def _():
        pltpu.sync_copy(acc_sh.at[pl.ds((NS - 1) * ROWS_PW, ROWS_LAST)],
                        out_hbm.at[cid, pl.ds((NS - 1) * ROWS_PW, ROWS_LAST)])


BM = 400


def _linear_body(parts_ref, w_ref, b_ref, o_ref):
    s = parts_ref[0] + parts_ref[1]
    o_ref[...] = lax.dot_general(
        s, w_ref[...], (((1,), (1,)), ((), ())),
        preferred_element_type=jnp.float32) + b_ref[...]


def _linear(parts, W, b2):
    return pl.pallas_call(
        _linear_body,
        grid=(N // BM,),
        in_specs=[
            pl.BlockSpec((NC, BM, D), lambda i: (0, i, 0)),
            pl.BlockSpec((D, D), lambda i: (0, 0)),
            pl.BlockSpec((1, D), lambda i: (0, 0)),
        ],
        out_specs=pl.BlockSpec((BM, D), lambda i: (i, 0)),
        out_shape=jax.ShapeDtypeStruct((N, D), jnp.float32),
    )(parts, W, b2)


def kernel(x, edge_index, W, b):
    src = edge_index[0]
    dst = edge_index[1]
    pad = E_PAD - E
    src_p = jnp.concatenate([src, jnp.zeros((pad,), jnp.int32)])
    dst_p = jnp.concatenate([dst, jnp.full((pad,), N, jnp.int32)])
    ecore = (jnp.arange(E_PAD, dtype=jnp.int32) >= R0E).astype(jnp.int32)
    src_p = src_p + ecore * N
    x2 = jnp.concatenate([x, x], axis=0)
    zeros = jnp.zeros((ROWS_PW, D), jnp.float32)
    parts = _gather_scatter_sum(x2, src_p, dst_p, zeros)
    h = _linear(parts, W, b.reshape(1, D))
    return h.reshape(1, N, D)


# speedup vs baseline: 10.6083x; 1.2191x over previous
import functools

import jax
import jax.numpy as jnp
from jax import lax
from jax.experimental import pallas as pl
from jax.experimental.pallas import tpu as pltpu
from jax.experimental.pallas import tpu_sc as plsc

N = 10000
D = 128
E = 320000

NC = 2
NS = 16
CH = 96
NFULL = E // CH
TAIL = E - NFULL * CH
TAIL_OFF = NFULL * CH
C0 = 2140
C1 = NFULL - C0
F0, R0X = C0 // NS, C0 % NS
F1, R1X = C1 // NS, C1 % NS
GMAX = (F0 + 1 + 3 + 7) // 8
N_ACC = 10008
ROWS_PW = 632
ROWS_LAST = N_ACC - 15 * ROWS_PW

_sc_mesh = plsc.VectorSubcoreMesh(core_axis_name="c", subcore_axis_name="s")


@functools.partial(
    pl.kernel,
    out_type=jax.ShapeDtypeStruct((NC, N_ACC, D), jnp.float32),
    mesh=_sc_mesh,
    scratch_types=[
        pltpu.VMEM((4, CH), jnp.int32),
        pltpu.VMEM((8, CH), jnp.int32),
        pltpu.VMEM((TAIL,), jnp.int32),
        pltpu.VMEM((TAIL,), jnp.int32),
        pltpu.VMEM((CH, D), jnp.float32),
        pltpu.VMEM((CH, D), jnp.float32),
        pltpu.VMEM((CH, D), jnp.float32),
        pltpu.VMEM((CH, D), jnp.float32),
        pltpu.VMEM_SHARED((N_ACC, D), jnp.float32),
        pltpu.SemaphoreType.DMA((4,)),
        pltpu.SemaphoreType.DMA((8,)),
        pltpu.SemaphoreType.DMA((4,)),
        pltpu.SemaphoreType.DMA((4,)),
    ],
)
def _gather_scatter_sum(x_hbm, ei_hbm, zeros_hbm, out_hbm,
                        sring, dring, tsrc, tdst, buf0, buf1, buf2, buf3,
                        acc_sh, issem, idsem, gsem, ssem):
    bufs = (buf0, buf1, buf2, buf3)
    cid = lax.axis_index("c")
    sid = lax.axis_index("s")
    nch = lax.select(cid == 0,
                     jnp.where(sid < R0X, F0 + 1, F0),
                     jnp.where(sid < R1X, F1 + 1, F1))
    bc0 = jnp.minimum(sid, R0X) * (F0 + 1) + jnp.maximum(sid - R0X, 0) * F0
    bc1 = C0 + jnp.minimum(sid, R1X) * (F1 + 1) + jnp.maximum(sid - R1X, 0) * F1
    ebase = lax.select(cid == 0, bc0, bc1) * CH

    r0 = sid * ROWS_PW

    @pl.when(sid < NS - 1)
    def _():
        pltpu.sync_copy(zeros_hbm, acc_sh.at[pl.ds(r0, ROWS_PW)])

    @pl.when(sid == NS - 1)
    def _():
        pltpu.sync_copy(zeros_hbm.at[pl.ds(0, ROWS_LAST)],
                        acc_sh.at[pl.ds((NS - 1) * ROWS_PW, ROWS_LAST)])

    plsc.subcore_barrier()

    def sidx_start(k, j):
        pltpu.async_copy(ei_hbm.at[pl.ds(ebase + k * CH, CH)], sring.at[j],
                         issem.at[j])

    def sidx_wait(k, j):
        pltpu.make_async_copy(ei_hbm.at[pl.ds(ebase + k * CH, CH)],
                              sring.at[j], issem.at[j]).wait()

    def didx_start(k, j):
        pltpu.async_copy(ei_hbm.at[pl.ds(E + ebase + k * CH, CH)], dring.at[j],
                         idsem.at[j])

    def didx_wait(k, j):
        pltpu.make_async_copy(ei_hbm.at[pl.ds(E + ebase + k * CH, CH)],
                              dring.at[j], idsem.at[j]).wait()

    def gather_start(j, r):
        pltpu.async_copy(x_hbm.at[sring.at[j]], bufs[r], gsem.at[r])

    def gather_wait(j, r):
        pltpu.make_async_copy(x_hbm.at[sring.at[j]], bufs[r],
                              gsem.at[r]).wait()

    def scatter_start(q, r):
        pltpu.async_copy(bufs[r], acc_sh.at[dring.at[q]], ssem.at[r],
                         add=True)

    def scatter_wait(q, r):
        pltpu.make_async_copy(bufs[r], acc_sh.at[dring.at[q]],
                              ssem.at[r]).wait()

    for j in range(4):
        sidx_start(j, j)
        didx_start(j, j)
    sidx_wait(0, 0)
    gather_start(0, 0)

    def body(g, carry):
        for u in range(8):
            k = g * 8 + u
            @pl.when(jnp.logical_and(k >= 3, k <= nch + 2))
            def _():
                scatter_wait((u - 3) % 8, (u - 3) % 4)
            @pl.when(k + 1 <= nch - 1)
            def _():
                sidx_wait(k + 1, (u + 1) % 4)
                gather_start((u + 1) % 4, (u + 1) % 4)
            @pl.when(k <= nch - 1)
            def _():
                gather_wait(u % 4, u % 4)
                didx_wait(k, u % 8)
                scatter_start(u % 8, u % 4)
            @pl.when(k + 4 <= nch - 1)
            def _():
                sidx_start(k + 4, u % 4)
                didx_start(k + 4, (u + 4) % 8)
        return carry

    lax.fori_loop(0, GMAX, body, 0)

    @pl.when(jnp.logical_and(cid == 1, sid == NS - 1))
    def _():
        pltpu.sync_copy(ei_hbm.at[pl.ds(TAIL_OFF, TAIL)], tsrc)
        pltpu.sync_copy(ei_hbm.at[pl.ds(E + TAIL_OFF, TAIL)], tdst)
        pltpu.async_copy(x_hbm.at[tsrc], buf0.at[pl.ds(0, TAIL)],
                         gsem.at[0]).wait()
        pltpu.sync_copy(buf0.at[pl.ds(0, TAIL)], acc_sh.at[tdst], add=True)

    plsc.subcore_barrier()

    @pl.when(sid < NS - 1)
    def _():
        pltpu.sync_copy(acc_sh.at[pl.ds(r0, ROWS_PW)],
                        out_hbm.at[cid, pl.ds(r0, ROWS_PW)])

    @pl.when(sid == NS - 1)
    def _():
        pltpu.sync_copy(acc_sh.at[pl.ds((NS - 1) * ROWS_PW, ROWS_LAST)],
                        out_hbm.at[cid, pl.ds((NS - 1) * ROWS_PW, ROWS_LAST)])


BM = 2000


def _linear_body(parts_ref, w_ref, b_ref, o_ref):
    s = parts_ref[0] + parts_ref[1]
    o_ref[...] = lax.dot_general(
        s, w_ref[...], (((1,), (1,)), ((), ())),
        preferred_element_type=jnp.float32) + b_ref[...]


def _linear(parts, W, b2):
    return pl.pallas_call(
        _linear_body,
        grid=(N // BM,),
        in_specs=[
            pl.BlockSpec((NC, BM, D), lambda i: (0, i, 0)),
            pl.BlockSpec((D, D), lambda i: (0, 0)),
            pl.BlockSpec((1, D), lambda i: (0, 0)),
        ],
        out_specs=pl.BlockSpec((BM, D), lambda i: (i, 0)),
        out_shape=jax.ShapeDtypeStruct((N, D), jnp.float32),
    )(parts, W, b2)


def kernel(x, edge_index, W, b):
    zeros = jnp.zeros((ROWS_PW, D), jnp.float32)
    parts = _gather_scatter_sum(x, edge_index.reshape(2 * E), zeros)
    h = _linear(parts, W, b.reshape(1, D))
    return h.reshape(1, N, D)


# speedup vs baseline: 11.9467x; 1.1262x over previous
import functools

import jax
import jax.numpy as jnp
from jax import lax
from jax.experimental import pallas as pl
from jax.experimental.pallas import tpu as pltpu
from jax.experimental.pallas import tpu_sc as plsc

N = 10000
D = 128
E = 320000

NC = 2
NS = 16
CH = 96
NFULL = E // CH
TAIL = E - NFULL * CH
TAIL_OFF = NFULL * CH
C0 = 1800
C1 = NFULL - C0
F0, R0X = C0 // NS, C0 % NS
F1, R1X = C1 // NS, C1 % NS
GMAX = (F0 + 1 + 3 + 7) // 8
N_ACC = 10008
ROWS_PW = 632
ROWS_LAST = N_ACC - 15 * ROWS_PW

_sc_mesh = plsc.VectorSubcoreMesh(core_axis_name="c", subcore_axis_name="s")


@functools.partial(
    pl.kernel,
    out_type=jax.ShapeDtypeStruct((NC, N_ACC, D), jnp.float32),
    mesh=_sc_mesh,
    scratch_types=[
        pltpu.VMEM((4, CH), jnp.int32),
        pltpu.VMEM((8, CH), jnp.int32),
        pltpu.VMEM((TAIL,), jnp.int32),
        pltpu.VMEM((TAIL,), jnp.int32),
        pltpu.VMEM((CH, D), jnp.float32),
        pltpu.VMEM((CH, D), jnp.float32),
        pltpu.VMEM((CH, D), jnp.float32),
        pltpu.VMEM((CH, D), jnp.float32),
        pltpu.VMEM_SHARED((N_ACC, D), jnp.float32),
        pltpu.SemaphoreType.DMA((4,)),
        pltpu.SemaphoreType.DMA((8,)),
        pltpu.SemaphoreType.DMA((4,)),
        pltpu.SemaphoreType.DMA((4,)),
    ],
)
def _gather_scatter_sum(x_hbm, ei_hbm, zeros_hbm, out_hbm,
                        sring, dring, tsrc, tdst, buf0, buf1, buf2, buf3,
                        acc_sh, issem, idsem, gsem, ssem):
    bufs = (buf0, buf1, buf2, buf3)
    cid = lax.axis_index("c")
    sid = lax.axis_index("s")
    nch = lax.select(cid == 0,
                     jnp.where(sid < R0X, F0 + 1, F0),
                     jnp.where(sid < R1X, F1 + 1, F1))
    bc0 = jnp.minimum(sid, R0X) * (F0 + 1) + jnp.maximum(sid - R0X, 0) * F0
    bc1 = C0 + jnp.minimum(sid, R1X) * (F1 + 1) + jnp.maximum(sid - R1X, 0) * F1
    ebase = lax.select(cid == 0, bc0, bc1) * CH

    r0 = sid * ROWS_PW

    @pl.when(sid < NS - 1)
    def _():
        pltpu.sync_copy(zeros_hbm, acc_sh.at[pl.ds(r0, ROWS_PW)])

    @pl.when(sid == NS - 1)
    def _():
        pltpu.sync_copy(zeros_hbm.at[pl.ds(0, ROWS_LAST)],
                        acc_sh.at[pl.ds((NS - 1) * ROWS_PW, ROWS_LAST)])

    plsc.subcore_barrier()

    def sidx_start(k, j):
        pltpu.async_copy(ei_hbm.at[pl.ds(ebase + k * CH, CH)], sring.at[j],
                         issem.at[j])

    def sidx_wait(k, j):
        pltpu.make_async_copy(ei_hbm.at[pl.ds(ebase + k * CH, CH)],
                              sring.at[j], issem.at[j]).wait()

    def didx_start(k, j):
        pltpu.async_copy(ei_hbm.at[pl.ds(E + ebase + k * CH, CH)], dring.at[j],
                         idsem.at[j])

    def didx_wait(k, j):
        pltpu.make_async_copy(ei_hbm.at[pl.ds(E + ebase + k * CH, CH)],
                              dring.at[j], idsem.at[j]).wait()

    def gather_start(j, r):
        pltpu.async_copy(x_hbm.at[sring.at[j]], bufs[r], gsem.at[r])

    def gather_wait(j, r):
        pltpu.make_async_copy(x_hbm.at[sring.at[j]], bufs[r],
                              gsem.at[r]).wait()

    def scatter_start(q, r):
        pltpu.async_copy(bufs[r], acc_sh.at[dring.at[q]], ssem.at[r],
                         add=True)

    def scatter_wait(q, r):
        pltpu.make_async_copy(bufs[r], acc_sh.at[dring.at[q]],
                              ssem.at[r]).wait()

    for j in range(4):
        sidx_start(j, j)
        didx_start(j, j)
    sidx_wait(0, 0)
    gather_start(0, 0)

    def body(g, carry):
        for u in range(8):
            k = g * 8 + u
            @pl.when(jnp.logical_and(k >= 3, k <= nch + 2))
            def _():
                scatter_wait((u - 3) % 8, (u - 3) % 4)
            @pl.when(k + 1 <= nch - 1)
            def _():
                sidx_wait(k + 1, (u + 1) % 4)
                gather_start((u + 1) % 4, (u + 1) % 4)
            @pl.when(k <= nch - 1)
            def _():
                gather_wait(u % 4, u % 4)
                didx_wait(k, u % 8)
                scatter_start(u % 8, u % 4)
            @pl.when(k + 4 <= nch - 1)
            def _():
                sidx_start(k + 4, u % 4)
                didx_start(k + 4, (u + 4) % 8)
        return carry

    lax.fori_loop(0, GMAX, body, 0)

    @pl.when(jnp.logical_and(cid == 1, sid == NS - 1))
    def _():
        pltpu.sync_copy(ei_hbm.at[pl.ds(TAIL_OFF, TAIL)], tsrc)
        pltpu.sync_copy(ei_hbm.at[pl.ds(E + TAIL_OFF, TAIL)], tdst)
        pltpu.async_copy(x_hbm.at[tsrc], buf0.at[pl.ds(0, TAIL)],
                         gsem.at[0]).wait()
        pltpu.sync_copy(buf0.at[pl.ds(0, TAIL)], acc_sh.at[tdst], add=True)

    plsc.subcore_barrier()

    @pl.when(sid < NS - 1)
    def _():
        pltpu.sync_copy(acc_sh.at[pl.ds(r0, ROWS_PW)],
                        out_hbm.at[cid, pl.ds(r0, ROWS_PW)])

    @pl.when(sid == NS - 1)
    def _():
        pltpu.sync_copy(acc_sh.at[pl.ds((NS - 1) * ROWS_PW, ROWS_LAST)],
                        out_hbm.at[cid, pl.ds((NS - 1) * ROWS_PW, ROWS_LAST)])


BM = 2000


def _linear_body(parts_ref, w_ref, b_ref, o_ref):
    s = parts_ref[0] + parts_ref[1]
    o_ref[...] = lax.dot_general(
        s, w_ref[...], (((1,), (1,)), ((), ())),
        preferred_element_type=jnp.float32) + b_ref[...]


def _linear(parts, W, b2):
    return pl.pallas_call(
        _linear_body,
        grid=(N // BM,),
        in_specs=[
            pl.BlockSpec((NC, BM, D), lambda i: (0, i, 0)),
            pl.BlockSpec((D, D), lambda i: (0, 0)),
            pl.BlockSpec((1, D), lambda i: (0, 0)),
        ],
        out_specs=pl.BlockSpec((BM, D), lambda i: (i, 0)),
        out_shape=jax.ShapeDtypeStruct((N, D), jnp.float32),
    )(parts, W, b2)


def kernel(x, edge_index, W, b):
    zeros = jnp.zeros((ROWS_PW, D), jnp.float32)
    parts = _gather_scatter_sum(x, edge_index.reshape(2 * E), zeros)
    h = _linear(parts, W, b.reshape(1, D))
    return h.reshape(1, N, D)


# speedup vs baseline: 13.6015x; 1.1385x over previous
import functools

import jax
import jax.numpy as jnp
from jax import lax
from jax.experimental import pallas as pl
from jax.experimental.pallas import tpu as pltpu
from jax.experimental.pallas import tpu_sc as plsc

N = 10000
D = 128
E = 320000

NC = 2
NS = 16
CH = 128
NFULL = E // CH
C0 = 1272
C1 = NFULL - C0
F0, R0X = C0 // NS, C0 % NS
F1, R1X = C1 // NS, C1 % NS
GMAX = (max(F0, F1) + 1 + 2 + 5) // 6
N_ACC = 10008
ROWS_PW = 632
ROWS_LAST = N_ACC - 15 * ROWS_PW

_sc_mesh = plsc.VectorSubcoreMesh(core_axis_name="c", subcore_axis_name="s")


@functools.partial(
    pl.kernel,
    out_type=jax.ShapeDtypeStruct((NC, N_ACC, D), jnp.float32),
    mesh=_sc_mesh,
    scratch_types=[
        pltpu.VMEM((2, CH), jnp.int32),
        pltpu.VMEM((2, CH), jnp.int32),
        pltpu.VMEM((2, CH), jnp.int32),
        pltpu.VMEM((2, CH), jnp.int32),
        pltpu.VMEM((2, CH), jnp.int32),
        pltpu.VMEM((2, CH), jnp.int32),
        pltpu.VMEM((CH, D), jnp.float32),
        pltpu.VMEM((CH, D), jnp.float32),
        pltpu.VMEM((CH, D), jnp.float32),
        pltpu.VMEM_SHARED((N_ACC, D), jnp.float32),
        pltpu.SemaphoreType.DMA((6,)),
        pltpu.SemaphoreType.DMA((3,)),
        pltpu.SemaphoreType.DMA((3,)),
    ],
)
def _gather_scatter_sum(x_hbm, ei_hbm, zeros_hbm, out_hbm,
                        c0r, c1r, c2r, c3r, c4r, c5r,
                        buf0, buf1, buf2,
                        acc_sh, icsem, gsem, ssem):
    crings = (c0r, c1r, c2r, c3r, c4r, c5r)
    bufs = (buf0, buf1, buf2)
    cid = lax.axis_index("c")
    sid = lax.axis_index("s")
    nch = lax.select(cid == 0,
                     jnp.where(sid < R0X, F0 + 1, F0),
                     jnp.where(sid < R1X, F1 + 1, F1))
    bc0 = jnp.minimum(sid, R0X) * (F0 + 1) + jnp.maximum(sid - R0X, 0) * F0
    bc1 = C0 + jnp.minimum(sid, R1X) * (F1 + 1) + jnp.maximum(sid - R1X, 0) * F1
    ebase = lax.select(cid == 0, bc0, bc1) * CH

    r0 = sid * ROWS_PW

    @pl.when(sid < NS - 1)
    def _():
        pltpu.sync_copy(zeros_hbm, acc_sh.at[pl.ds(r0, ROWS_PW)])

    @pl.when(sid == NS - 1)
    def _():
        pltpu.sync_copy(zeros_hbm.at[pl.ds(0, ROWS_LAST)],
                        acc_sh.at[pl.ds((NS - 1) * ROWS_PW, ROWS_LAST)])

    plsc.subcore_barrier()

    def cidx_start(k, j):
        pltpu.async_copy(ei_hbm.at[:, pl.ds(ebase + k * CH, CH)], crings[j],
                         icsem.at[j])

    def cidx_wait(k, j):
        pltpu.make_async_copy(ei_hbm.at[:, pl.ds(ebase + k * CH, CH)],
                              crings[j], icsem.at[j]).wait()

    def gather_start(j, r):
        pltpu.async_copy(x_hbm.at[crings[j].at[0]], bufs[r], gsem.at[r])

    def gather_wait(j, r):
        pltpu.make_async_copy(x_hbm.at[crings[j].at[0]], bufs[r],
                              gsem.at[r]).wait()

    def scatter_start(j, r):
        pltpu.async_copy(bufs[r], acc_sh.at[crings[j].at[1]], ssem.at[r],
                         add=True)

    def scatter_wait(j, r):
        pltpu.make_async_copy(bufs[r], acc_sh.at[crings[j].at[1]],
                              ssem.at[r]).wait()

    for j in range(4):
        cidx_start(j, j)
    cidx_wait(0, 0)
    gather_start(0, 0)

    def body(g, carry):
        for u in range(6):
            k = g * 6 + u
            @pl.when(jnp.logical_and(k >= 2, k <= nch + 1))
            def _():
                scatter_wait((u - 2) % 6, (u - 2) % 3)
            @pl.when(k + 1 <= nch - 1)
            def _():
                cidx_wait(k + 1, (u + 1) % 6)
                gather_start((u + 1) % 6, (u + 1) % 3)
            @pl.when(k <= nch - 1)
            def _():
                gather_wait(u % 6, u % 3)
                scatter_start(u % 6, u % 3)
            @pl.when(k + 4 <= nch - 1)
            def _():
                cidx_start(k + 4, (u + 4) % 6)
        return carry

    lax.fori_loop(0, GMAX, body, 0)
    plsc.subcore_barrier()

    @pl.when(sid < NS - 1)
    def _():
        pltpu.sync_copy(acc_sh.at[pl.ds(r0, ROWS_PW)],
                        out_hbm.at[cid, pl.ds(r0, ROWS_PW)])

    @pl.when(sid == NS - 1)
    def _():
        pltpu.sync_copy(acc_sh.at[pl.ds((NS - 1) * ROWS_PW, ROWS_LAST)],
                        out_hbm.at[cid, pl.ds((NS - 1) * ROWS_PW, ROWS_LAST)])


BM = 2000


def _linear_body(parts_ref, w_ref, b_ref, o_ref):
    s = parts_ref[0] + parts_ref[1]
    o_ref[...] = lax.dot_general(
        s, w_ref[...], (((1,), (1,)), ((), ())),
        preferred_element_type=jnp.float32) + b_ref[...]


def _linear(parts, W, b2):
    return pl.pallas_call(
        _linear_body,
        grid=(N // BM,),
        in_specs=[
            pl.BlockSpec((NC, BM, D), lambda i: (0, i, 0)),
            pl.BlockSpec((D, D), lambda i: (0, 0)),
            pl.BlockSpec((1, D), lambda i: (0, 0)),
        ],
        out_specs=pl.BlockSpec((BM, D), lambda i: (i, 0)),
        out_shape=jax.ShapeDtypeStruct((N, D), jnp.float32),
    )(parts, W, b2)


def kernel(x, edge_index, W, b):
    zeros = jnp.zeros((ROWS_PW, D), jnp.float32)
    parts = _gather_scatter_sum(x, edge_index, zeros)
    h = _linear(parts, W, b.reshape(1, D))
    return h.reshape(1, N, D)


# speedup vs baseline: 13.7124x; 1.0082x over previous
import functools

import jax
import jax.numpy as jnp
from jax import lax
from jax.experimental import pallas as pl
from jax.experimental.pallas import tpu as pltpu
from jax.experimental.pallas import tpu_sc as plsc

N = 10000
D = 128
E = 320000

NC = 2
NS = 16
CH = 128
NFULL = E // CH
C0 = 1253
C1 = NFULL - C0
F0, R0X = C0 // NS, C0 % NS
F1, R1X = C1 // NS, C1 % NS
GMAX = (max(F0, F1) + 1 + 2 + 5) // 6
N_ACC = 10008
ROWS_PW = 632
ROWS_LAST = N_ACC - 15 * ROWS_PW

_sc_mesh = plsc.VectorSubcoreMesh(core_axis_name="c", subcore_axis_name="s")


@functools.partial(
    pl.kernel,
    out_type=jax.ShapeDtypeStruct((NC, N_ACC, D), jnp.float32),
    mesh=_sc_mesh,
    scratch_types=[
        pltpu.VMEM((2, CH), jnp.int32),
        pltpu.VMEM((2, CH), jnp.int32),
        pltpu.VMEM((2, CH), jnp.int32),
        pltpu.VMEM((2, CH), jnp.int32),
        pltpu.VMEM((2, CH), jnp.int32),
        pltpu.VMEM((2, CH), jnp.int32),
        pltpu.VMEM((CH, D), jnp.float32),
        pltpu.VMEM((CH, D), jnp.float32),
        pltpu.VMEM((CH, D), jnp.float32),
        pltpu.VMEM_SHARED((N_ACC, D), jnp.float32),
        pltpu.SemaphoreType.DMA((6,)),
        pltpu.SemaphoreType.DMA((3,)),
        pltpu.SemaphoreType.DMA((3,)),
    ],
)
def _gather_scatter_sum(x_hbm, ei_hbm, zeros_hbm, out_hbm,
                        c0r, c1r, c2r, c3r, c4r, c5r,
                        buf0, buf1, buf2,
                        acc_sh, icsem, gsem, ssem):
    crings = (c0r, c1r, c2r, c3r, c4r, c5r)
    bufs = (buf0, buf1, buf2)
    cid = lax.axis_index("c")
    sid = lax.axis_index("s")
    nch = lax.select(cid == 0,
                     jnp.where(sid < R0X, F0 + 1, F0),
                     jnp.where(sid < R1X, F1 + 1, F1))
    bc0 = jnp.minimum(sid, R0X) * (F0 + 1) + jnp.maximum(sid - R0X, 0) * F0
    bc1 = C0 + jnp.minimum(sid, R1X) * (F1 + 1) + jnp.maximum(sid - R1X, 0) * F1
    ebase = lax.select(cid == 0, bc0, bc1) * CH

    r0 = sid * ROWS_PW

    def cidx_start(k, j):
        pltpu.async_copy(ei_hbm.at[:, pl.ds(ebase + k * CH, CH)], crings[j],
                         icsem.at[j])

    def cidx_wait(k, j):
        pltpu.make_async_copy(ei_hbm.at[:, pl.ds(ebase + k * CH, CH)],
                              crings[j], icsem.at[j]).wait()

    def gather_start(j, r):
        pltpu.async_copy(x_hbm.at[crings[j].at[0]], bufs[r], gsem.at[r])

    def gather_wait(j, r):
        pltpu.make_async_copy(x_hbm.at[crings[j].at[0]], bufs[r],
                              gsem.at[r]).wait()

    def scatter_start(j, r):
        pltpu.async_copy(bufs[r], acc_sh.at[crings[j].at[1]], ssem.at[r],
                         add=True)

    def scatter_wait(j, r):
        pltpu.make_async_copy(bufs[r], acc_sh.at[crings[j].at[1]],
                              ssem.at[r]).wait()

    for j in range(4):
        cidx_start(j, j)
    cidx_wait(0, 0)
    gather_start(0, 0)

    @pl.when(sid < NS - 1)
    def _():
        pltpu.sync_copy(zeros_hbm, acc_sh.at[pl.ds(r0, ROWS_PW)])

    @pl.when(sid == NS - 1)
    def _():
        pltpu.sync_copy(zeros_hbm.at[pl.ds(0, ROWS_LAST)],
                        acc_sh.at[pl.ds((NS - 1) * ROWS_PW, ROWS_LAST)])

    plsc.subcore_barrier()

    def body(g, carry):
        for u in range(6):
            k = g * 6 + u
            @pl.when(jnp.logical_and(k >= 2, k <= nch + 1))
            def _():
                scatter_wait((u - 2) % 6, (u - 2) % 3)
            @pl.when(k + 1 <= nch - 1)
            def _():
                cidx_wait(k + 1, (u + 1) % 6)
                gather_start((u + 1) % 6, (u + 1) % 3)
            @pl.when(k <= nch - 1)
            def _():
                gather_wait(u % 6, u % 3)
                scatter_start(u % 6, u % 3)
            @pl.when(k + 4 <= nch - 1)
            def _():
                cidx_start(k + 4, (u + 4) % 6)
        return carry

    lax.fori_loop(0, GMAX, body, 0)
    plsc.subcore_barrier()

    @pl.when(sid < NS - 1)
    def _():
        pltpu.sync_copy(acc_sh.at[pl.ds(r0, ROWS_PW)],
                        out_hbm.at[cid, pl.ds(r0, ROWS_PW)])

    @pl.when(sid == NS - 1)
    def _():
        pltpu.sync_copy(acc_sh.at[pl.ds((NS - 1) * ROWS_PW, ROWS_LAST)],
                        out_hbm.at[cid, pl.ds((NS - 1) * ROWS_PW, ROWS_LAST)])


BM = 2000


def _linear_body(parts_ref, w_ref, b_ref, o_ref):
    s = parts_ref[0] + parts_ref[1]
    o_ref[...] = lax.dot_general(
        s, w_ref[...], (((1,), (1,)), ((), ())),
        preferred_element_type=jnp.float32) + b_ref[...]


def _linear(parts, W, b2):
    return pl.pallas_call(
        _linear_body,
        grid=(N // BM,),
        in_specs=[
            pl.BlockSpec((NC, BM, D), lambda i: (0, i, 0)),
            pl.BlockSpec((D, D), lambda i: (0, 0)),
            pl.BlockSpec((1, D), lambda i: (0, 0)),
        ],
        out_specs=pl.BlockSpec((BM, D), lambda i: (i, 0)),
        out_shape=jax.ShapeDtypeStruct((N, D), jnp.float32),
    )(parts, W, b2)


def kernel(x, edge_index, W, b):
    zeros = jnp.zeros((ROWS_PW, D), jnp.float32)
    parts = _gather_scatter_sum(x, edge_index, zeros)
    h = _linear(parts, W, b.reshape(1, D))
    return h.reshape(1, N, D)


# speedup vs baseline: 13.7952x; 1.0060x over previous
import functools

import jax
import jax.numpy as jnp
from jax import lax
from jax.experimental import pallas as pl
from jax.experimental.pallas import tpu as pltpu
from jax.experimental.pallas import tpu_sc as plsc

N = 10000
D = 128
E = 320000

NC = 2
NS = 16
CH = 128
NFULL = E // CH
C0 = 1232
C1 = NFULL - C0
F0, R0X = C0 // NS, C0 % NS
F1, R1X = C1 // NS, C1 % NS
GMAX = (max(F0, F1) + 1 + 2 + 5) // 6
N_ACC = 10008
ROWS_PW = 632
ROWS_LAST = N_ACC - 15 * ROWS_PW

_sc_mesh = plsc.VectorSubcoreMesh(core_axis_name="c", subcore_axis_name="s")


@functools.partial(
    pl.kernel,
    out_type=jax.ShapeDtypeStruct((NC, N_ACC, D), jnp.float32),
    mesh=_sc_mesh,
    scratch_types=[
        pltpu.VMEM((2, CH), jnp.int32),
        pltpu.VMEM((2, CH), jnp.int32),
        pltpu.VMEM((2, CH), jnp.int32),
        pltpu.VMEM((2, CH), jnp.int32),
        pltpu.VMEM((2, CH), jnp.int32),
        pltpu.VMEM((2, CH), jnp.int32),
        pltpu.VMEM((CH, D), jnp.float32),
        pltpu.VMEM((CH, D), jnp.float32),
        pltpu.VMEM((CH, D), jnp.float32),
        pltpu.VMEM_SHARED((N_ACC, D), jnp.float32),
        pltpu.SemaphoreType.DMA((6,)),
        pltpu.SemaphoreType.DMA((3,)),
        pltpu.SemaphoreType.DMA((3,)),
    ],
)
def _gather_scatter_sum(x_hbm, ei_hbm, zeros_hbm, out_hbm,
                        c0r, c1r, c2r, c3r, c4r, c5r,
                        buf0, buf1, buf2,
                        acc_sh, icsem, gsem, ssem):
    crings = (c0r, c1r, c2r, c3r, c4r, c5r)
    bufs = (buf0, buf1, buf2)
    cid = lax.axis_index("c")
    sid = lax.axis_index("s")
    nch = lax.select(cid == 0,
                     jnp.where(sid < R0X, F0 + 1, F0),
                     jnp.where(sid < R1X, F1 + 1, F1))
    bc0 = jnp.minimum(sid, R0X) * (F0 + 1) + jnp.maximum(sid - R0X, 0) * F0
    bc1 = C0 + jnp.minimum(sid, R1X) * (F1 + 1) + jnp.maximum(sid - R1X, 0) * F1
    ebase = lax.select(cid == 0, bc0, bc1) * CH

    r0 = sid * ROWS_PW

    def cidx_start(k, j):
        pltpu.async_copy(ei_hbm.at[:, pl.ds(ebase + k * CH, CH)], crings[j],
                         icsem.at[j])

    def cidx_wait(k, j):
        pltpu.make_async_copy(ei_hbm.at[:, pl.ds(ebase + k * CH, CH)],
                              crings[j], icsem.at[j]).wait()

    def gather_start(j, r):
        pltpu.async_copy(x_hbm.at[crings[j].at[0]], bufs[r], gsem.at[r])

    def gather_wait(j, r):
        pltpu.make_async_copy(x_hbm.at[crings[j].at[0]], bufs[r],
                              gsem.at[r]).wait()

    def scatter_start(j, r):
        pltpu.async_copy(bufs[r], acc_sh.at[crings[j].at[1]], ssem.at[r],
                         add=True)

    def scatter_wait(j, r):
        pltpu.make_async_copy(bufs[r], acc_sh.at[crings[j].at[1]],
                              ssem.at[r]).wait()

    for j in range(4):
        cidx_start(j, j)
    cidx_wait(0, 0)
    gather_start(0, 0)

    @pl.when(sid < NS - 1)
    def _():
        pltpu.sync_copy(zeros_hbm, acc_sh.at[pl.ds(r0, ROWS_PW)])

    @pl.when(sid == NS - 1)
    def _():
        pltpu.sync_copy(zeros_hbm.at[pl.ds(0, ROWS_LAST)],
                        acc_sh.at[pl.ds((NS - 1) * ROWS_PW, ROWS_LAST)])

    plsc.subcore_barrier()

    def body(g, carry):
        for u in range(6):
            k = g * 6 + u
            @pl.when(jnp.logical_and(k >= 2, k <= nch + 1))
            def _():
                scatter_wait((u - 2) % 6, (u - 2) % 3)
            @pl.when(k + 1 <= nch - 1)
            def _():
                cidx_wait(k + 1, (u + 1) % 6)
                gather_start((u + 1) % 6, (u + 1) % 3)
            @pl.when(k <= nch - 1)
            def _():
                gather_wait(u % 6, u % 3)
                scatter_start(u % 6, u % 3)
            @pl.when(k + 4 <= nch - 1)
            def _():
                cidx_start(k + 4, (u + 4) % 6)
        return carry

    lax.fori_loop(0, GMAX, body, 0)
    plsc.subcore_barrier()

    @pl.when(sid < NS - 1)
    def _():
        pltpu.sync_copy(acc_sh.at[pl.ds(r0, ROWS_PW)],
                        out_hbm.at[cid, pl.ds(r0, ROWS_PW)])

    @pl.when(sid == NS - 1)
    def _():
        pltpu.sync_copy(acc_sh.at[pl.ds((NS - 1) * ROWS_PW, ROWS_LAST)],
                        out_hbm.at[cid, pl.ds((NS - 1) * ROWS_PW, ROWS_LAST)])


BM = 2000


def _linear_body(parts_ref, w_ref, b_ref, o_ref):
    s = parts_ref[0] + parts_ref[1]
    o_ref[...] = lax.dot_general(
        s, w_ref[...], (((1,), (1,)), ((), ())),
        preferred_element_type=jnp.float32) + b_ref[...]


def _linear(parts, W, b2):
    return pl.pallas_call(
        _linear_body,
        grid=(N // BM,),
        in_specs=[
            pl.BlockSpec((NC, BM, D), lambda i: (0, i, 0)),
            pl.BlockSpec((D, D), lambda i: (0, 0)),
            pl.BlockSpec((1, D), lambda i: (0, 0)),
        ],
        out_specs=pl.BlockSpec((BM, D), lambda i: (i, 0)),
        out_shape=jax.ShapeDtypeStruct((N, D), jnp.float32),
    )(parts, W, b2)


def kernel(x, edge_index, W, b):
    zeros = jnp.zeros((ROWS_PW, D), jnp.float32)
    parts = _gather_scatter_sum(x, edge_index, zeros)
    h = _linear(parts, W, b.reshape(1, D))
    return h.reshape(1, N, D)
